# baseline (speedup 1.0000x reference)
import functools

import jax
import jax.numpy as jnp
from jax import lax
from jax.experimental import pallas as pl
from jax.experimental.pallas import tpu as pltpu
from jax.experimental.pallas import tpu_sc as plsc

_N = 10000
_E = 320000
_D = 128
_H = 128
_M = 3
_DEPTH = 3
_H3 = _H // 3
_H9 = _H3 // 3

_NC = 2
_NS = 16
_NW = _NC * _NS
_EPT = _E // _NW
_K = 128
_EPTP = 10240
_NCHUNK = _EPTP // _K
_NPAD = 10240
_RPT = _NPAD // _NS

@functools.partial(
    pl.kernel,
    out_type=jax.ShapeDtypeStruct((_M * 2 * _NPAD, _H), jnp.float32),
    mesh=plsc.VectorSubcoreMesh(core_axis_name="c", subcore_axis_name="s"),
    scratch_types=[
        pltpu.VMEM((2, _K), jnp.int32),
        pltpu.VMEM((2, _K), jnp.int32),
        pltpu.VMEM((2, _K, _H), jnp.float32),
        pltpu.VMEM_SHARED((_NPAD, _H), jnp.float32),
        ((pltpu.SemaphoreType.DMA, pltpu.SemaphoreType.DMA),
         (pltpu.SemaphoreType.DMA, pltpu.SemaphoreType.DMA),
         (pltpu.SemaphoreType.DMA, pltpu.SemaphoreType.DMA)),
    ],
)
def _sc_segsum(h_hbm, srcf_hbm, dstf_hbm, zeros_hbm, agg_hbm,
               srcr, dstr, rows_v, acc, sems):
    gsem, ssem, dsem = sems
    c = lax.axis_index("c")
    s = lax.axis_index("s")
    wid = c * _NS + s
    base_d = wid * _NCHUNK

    def gather(b):
        pltpu.async_copy(h_hbm.at[srcr.at[b]], rows_v.at[b], gsem[b])

    def gwait(b):
        pltpu.make_async_copy(h_hbm.at[pl.ds(0, _K)], rows_v.at[b],
                              gsem[b]).wait()

    def scat(b):
        pltpu.sync_copy(rows_v.at[b], acc.at[dstr.at[b]], add=True)

    def ifetch(base_s, j, b):
        pltpu.async_copy(srcf_hbm.at[base_s + j], srcr.at[b], ssem[b])
        pltpu.async_copy(dstf_hbm.at[base_d + j], dstr.at[b], dsem[b])

    def iwait(b):
        pltpu.make_async_copy(srcf_hbm.at[0], srcr.at[b], ssem[b]).wait()
        pltpu.make_async_copy(dstf_hbm.at[0], dstr.at[b], dsem[b]).wait()

    for m in range(_M):
        base_s = (m * _NW + wid) * _NCHUNK
        ifetch(base_s, 0, 0)
        ifetch(base_s, 1, 1)
        pltpu.sync_copy(zeros_hbm, acc.at[pl.ds(s * _RPT, _RPT)])
        plsc.subcore_barrier()
        iwait(0)
        gather(0)

        def group(g, carry):
            gwait(0)
            iwait(1)
            gather(1)
            scat(0)
            ifetch(base_s, 2 * g + 2, 0)
            gwait(1)
            iwait(0)
            gather(0)
            scat(1)
            ifetch(base_s, 2 * g + 3, 1)
            return carry

        lax.fori_loop(0, _NCHUNK // 2 - 1, group, 0)
        gwait(0)
        iwait(1)
        gather(1)
        scat(0)
        gwait(1)
        scat(1)

        plsc.subcore_barrier()
        row0 = (2 * m + c) * _NPAD + s * _RPT
        pltpu.sync_copy(acc.at[pl.ds(s * _RPT, _RPT)],
                        agg_hbm.at[pl.ds(row0, _RPT)])


_BN = 1000
_NB = _N // _BN


def _h0_body(x_ref, wi_ref, out_ref):
    x = x_ref[...]
    for m in range(_M):
        out_ref[m] = jnp.maximum(lax.dot(x, wi_ref[m]), 0.0)


_h0_call = pl.pallas_call(
    _h0_body,
    grid=(_NB,),
    in_specs=[
        pl.BlockSpec((_BN, _D), lambda i: (i, 0)),
        pl.BlockSpec((_M, _D, _H), lambda i: (0, 0, 0)),
    ],
    out_specs=pl.BlockSpec((_M, _BN, _H), lambda i: (0, i, 0)),
    out_shape=jax.ShapeDtypeStruct((_M, _N, _H), jnp.float32),
)


def _upd_body(h0_ref, agg_ref, wh_ref, out_ref):
    for m in range(_M):
        a = agg_ref[m, 0] + agg_ref[m, 1]
        out_ref[m] = jnp.maximum(
            h0_ref[m] + lax.dot(a, wh_ref[m]), 0.0)


_upd_call = pl.pallas_call(
    _upd_body,
    grid=(_NB,),
    in_specs=[
        pl.BlockSpec((_M, _BN, _H), lambda i: (0, i, 0)),
        pl.BlockSpec((_M, 2, _BN, _H), lambda i: (0, 0, i, 0)),
        pl.BlockSpec((_M, _H, _H), lambda i: (0, 0, 0)),
    ],
    out_specs=pl.BlockSpec((_M, _BN, _H), lambda i: (0, i, 0)),
    out_shape=jax.ShapeDtypeStruct((_M, _N, _H), jnp.float32),
)


def _readout_body(x_ref, h_ref, wo_ref, bo_ref, w1_ref, b1_ref, w2_ref,
                  b2_ref, cw1_ref, cb1_ref, cw2_ref, cb2_ref, cw3_ref,
                  cb3_ref, out_ref, acc_ref):
    i = pl.program_id(0)

    @pl.when(i == 0)
    def _():
        acc_ref[...] = jnp.zeros_like(acc_ref)

    x = x_ref[...]
    for m in range(_M):
        ah = jnp.maximum(
            lax.dot(x, wo_ref[m, :_D, :])
            + lax.dot(h_ref[m], wo_ref[m, _D:, :])
            + bo_ref[m][None, :], 0.0)
        acc_ref[m, :] = acc_ref[m, :] + jnp.sum(ah, axis=0)

    @pl.when(i == _NB - 1)
    def _():
        def rb(v):
            return v.astype(jnp.bfloat16).astype(jnp.float32)

        total = 0.0
        for m in range(_M):
            e = rb(acc_ref[m, :] * (1.0 / _N))
            t = jnp.maximum(
                jnp.sum(e[:, None] * rb(w1_ref[m]), axis=0) + b1_ref[m], 0.0)
            temp = jnp.sum(rb(t) * rb(w2_ref[m])) + b2_ref[m]
            z = jnp.maximum(
                jnp.sum(e[:, None] * rb(cw1_ref[m]), axis=0) + cb1_ref[m], 0.0)
            z2 = jnp.maximum(
                jnp.sum(rb(z)[:, None] * rb(cw2_ref[m]), axis=0) + cb2_ref[m], 0.0)
            coef = jnp.sum(rb(z2) * rb(cw3_ref[m])) + cb3_ref[m]
            total = total + temp * coef
        out_ref[...] = jnp.reshape(total, (1, 1))


_readout_call = pl.pallas_call(
    _readout_body,
    grid=(_NB,),
    in_specs=[
        pl.BlockSpec((_BN, _D), lambda i: (i, 0)),
        pl.BlockSpec((_M, _BN, _H), lambda i: (0, i, 0)),
        pl.BlockSpec((_M, _D + _H, _H), lambda i: (0, 0, 0)),
        pl.BlockSpec((_M, _H), lambda i: (0, 0)),
        pl.BlockSpec((_M, _H, _H), lambda i: (0, 0, 0)),
        pl.BlockSpec((_M, _H), lambda i: (0, 0)),
        pl.BlockSpec((_M, _H), lambda i: (0, 0)),
        pl.BlockSpec((_M,), lambda i: (0,)),
        pl.BlockSpec((_M, _H, _H3), lambda i: (0, 0, 0)),
        pl.BlockSpec((_M, _H3), lambda i: (0, 0)),
        pl.BlockSpec((_M, _H3, _H9), lambda i: (0, 0, 0)),
        pl.BlockSpec((_M, _H9), lambda i: (0, 0)),
        pl.BlockSpec((_M, _H9), lambda i: (0, 0)),
        pl.BlockSpec((_M,), lambda i: (0,)),
    ],
    out_specs=pl.BlockSpec((1, 1), lambda i: (0, 0)),
    out_shape=jax.ShapeDtypeStruct((1, 1), jnp.float32),
    scratch_shapes=[pltpu.VMEM((_M, _H), jnp.float32)],
)


def kernel(x, edge_index, W_i, W_h, W_o, b_o, ffn_W1, ffn_b1, ffn_W2, ffn_b2,
           c_W1, c_b1, c_W2, c_b2, c_W3, c_b3):
    src = edge_index[0]
    dst = edge_index[1]
    npad_e = _EPTP - _EPT
    src_r = jnp.concatenate(
        [src.reshape(_NW, _EPT),
         jnp.zeros((_NW, npad_e), jnp.int32)], axis=1)
    dst_r = jnp.concatenate(
        [dst.reshape(_NW, _EPT),
         jnp.full((_NW, npad_e), _N, jnp.int32)], axis=1)
    offs = (jnp.arange(_M, dtype=jnp.int32) * _N)[:, None, None]
    src_m = (src_r[None] + offs).reshape(_M * _NW * _NCHUNK, _K)
    dst_r = dst_r.reshape(_NW * _NCHUNK, _K)
    zeros = jnp.zeros((_RPT, _H), jnp.float32)

    h0 = _h0_call(x, W_i)
    h = h0
    for _ in range(_DEPTH):
        agg_flat = _sc_segsum(h.reshape(_M * _N, _H), src_m, dst_r, zeros)
        agg = agg_flat.reshape(_M, 2, _NPAD, _H)
        h = _upd_call(h0, agg, W_h)
    out = _readout_call(x, h, W_o, b_o, ffn_W1, ffn_b1, ffn_W2[..., 0],
                        ffn_b2[..., 0], c_W1, c_b1, c_W2, c_b2, c_W3[..., 0],
                        c_b3[..., 0])
    return out
